# Initial kernel scaffold; baseline (speedup 1.0000x reference)
import jax
import jax.numpy as jnp
from jax.experimental import pallas as pl


def kernel(obj_vecs, rel_vecs, edge_index, W_obj, b_obj, W_rel, b_rel):
    raise NotImplementedError("write your pallas kernel here")


# speedup vs baseline: 11.6019x; 11.6019x over previous
import functools

import jax
import jax.numpy as jnp
from jax import lax
from jax.experimental import pallas as pl
from jax.experimental.pallas import tpu as pltpu
from jax.experimental.pallas import tpu_sc as plsc

N = 10000
D = 128
E = 320000

NC = 2
NS = 16

A_B = 125
A_CHUNKS = E // (NC * NS * A_B)
NP = 10240
NP_TILE = NP // NS


def _deg_body(dst_hbm, out_hbm, idx_v, ones_v, zero_v, buf_v, deg_sh, sem):
    c = lax.axis_index("c")
    s = lax.axis_index("s")
    for j in range(8):
        ones_v[pl.ds(16 * j, 16)] = jnp.full((16,), 1.0, jnp.float32)
        zero_v[pl.ds(16 * j, 16)] = jnp.zeros((16,), jnp.float32)
    for j in range(NP_TILE // 128):
        pltpu.sync_copy(zero_v, deg_sh.at[pl.ds(s * NP_TILE + j * 128, 128)])
    plsc.subcore_barrier()
    pltpu.sync_copy(dst_hbm.at[c, s], idx_v)

    def body(j, carry):
        pltpu.sync_copy(ones_v.at[pl.ds(0, A_B)], deg_sh.at[idx_v.at[j]], add=True)
        return carry

    lax.fori_loop(0, A_CHUNKS, body, 0)
    plsc.subcore_barrier()
    pltpu.sync_copy(deg_sh.at[pl.ds(s * NP_TILE, NP_TILE)], buf_v)
    pltpu.sync_copy(buf_v, out_hbm.at[pl.ds(c * NP + s * NP_TILE, NP_TILE)])


_deg_kernel = functools.partial(
    pl.kernel,
    out_type=jax.ShapeDtypeStruct((NC * NP,), jnp.float32),
    mesh=plsc.VectorSubcoreMesh(core_axis_name="c", subcore_axis_name="s"),
    scratch_types=[
        pltpu.VMEM((A_CHUNKS, A_B), jnp.int32),
        pltpu.VMEM((128,), jnp.float32),
        pltpu.VMEM((128,), jnp.float32),
        pltpu.VMEM((NP_TILE,), jnp.float32),
        pltpu.VMEM_SHARED((NP,), jnp.float32),
        pltpu.SemaphoreType.DMA,
    ],
)(_deg_body)


C_B = 128
C_CHUNKS = 157
E_TILE = C_CHUNKS * C_B
E_PAD = NS * E_TILE
HN = N // 2
ACC_ROWS = HN + 8

def _scatter_body(y_hbm, src_hbm, dst_hbm, out_hbm,
                  srcv, dstv, dloc, rows, zbuf, acc_sh, sem):
    c = lax.axis_index("c")
    s = lax.axis_index("s")

    def zb(i, carry):
        for j in range(D // 16):
            zbuf[i, pl.ds(16 * j, 16)] = jnp.zeros((16,), jnp.float32)
        return carry

    lax.fori_loop(0, 16, zb, 0)
    pltpu.sync_copy(src_hbm.at[s], srcv)
    pltpu.sync_copy(dst_hbm.at[s], dstv)

    for h in range(2):
        lo = h * HN

        def lb(j, carry):
            for k in range(C_B // 16):
                dv = dstv[j, pl.ds(16 * k, 16)]
                m = (dv >= lo) & (dv < lo + HN)
                dloc[j, pl.ds(16 * k, 16)] = jnp.where(
                    m, dv - lo, jnp.int32(HN))
            return carry

        lax.fori_loop(0, C_CHUNKS, lb, 0)

        def zc(i, carry):
            k = s + 16 * i

            @pl.when(k < ACC_ROWS // 16)
            def _():
                pltpu.sync_copy(zbuf, acc_sh.at[pl.ds(16 * k, 16)])

            return carry

        lax.fori_loop(0, (ACC_ROWS // 16 + NS - 1) // NS, zc, 0)
        plsc.subcore_barrier()

        def ebody(j, carry):
            pltpu.async_copy(y_hbm.at[c].at[srcv.at[j]], rows, sem).wait()
            pltpu.sync_copy(rows, acc_sh.at[dloc.at[j]], add=True)
            return carry

        lax.fori_loop(0, C_CHUNKS, ebody, 0)
        plsc.subcore_barrier()

        def oc(i, carry):
            k = s + 16 * i

            @pl.when(k < HN // 8)
            def _():
                pltpu.sync_copy(acc_sh.at[pl.ds(8 * k, 8)], rows.at[pl.ds(0, 8)])
                pltpu.sync_copy(rows.at[pl.ds(0, 8)],
                                out_hbm.at[c].at[pl.ds(lo + 8 * k, 8)])

            return carry

        lax.fori_loop(0, (HN // 8 + NS - 1) // NS, oc, 0)
        if h == 0:
            plsc.subcore_barrier()


_scatter_kernel = functools.partial(
    pl.kernel,
    out_type=jax.ShapeDtypeStruct((NC, N, D), jnp.float32),
    mesh=plsc.VectorSubcoreMesh(core_axis_name="c", subcore_axis_name="s"),
    scratch_types=[
        pltpu.VMEM((C_CHUNKS, C_B), jnp.int32),
        pltpu.VMEM((C_CHUNKS, C_B), jnp.int32),
        pltpu.VMEM((C_CHUNKS, C_B), jnp.int32),
        pltpu.VMEM((C_B, D), jnp.float32),
        pltpu.VMEM((16, D), jnp.float32),
        pltpu.VMEM_SHARED((ACC_ROWS, D), jnp.float32),
        pltpu.SemaphoreType.DMA,
    ],
)(_scatter_body)


def _mm_body(obj_ref, rel_ref, wo_ref, wr_ref, dinv_ref, y_ref):
    dinv = dinv_ref[...]
    y_ref[0] = lax.dot_general(obj_ref[...], wo_ref[...],
                               (((1,), (1,)), ((), ())),
                               preferred_element_type=jnp.float32) * dinv
    y_ref[1] = lax.dot_general(rel_ref[...], wr_ref[...],
                               (((1,), (1,)), ((), ())),
                               preferred_element_type=jnp.float32) * dinv


def _mm(obj_vecs, rel_vecs, W_obj, W_rel, dinv2d):
    return pl.pallas_call(
        _mm_body,
        out_shape=jax.ShapeDtypeStruct((NC, N, D), jnp.float32),
    )(obj_vecs, rel_vecs, W_obj, W_rel, dinv2d)


def _fin_body(acc_ref, y_ref, dinv_ref, bo_ref, br_ref, oo_ref, or_ref):
    dinv = dinv_ref[...]
    oo_ref[...] = jnp.maximum(
        (acc_ref[0] + y_ref[0]) * dinv + bo_ref[...][None, :], 0.0)
    or_ref[...] = jnp.maximum(
        (acc_ref[1] + y_ref[1]) * dinv + br_ref[...][None, :], 0.0)


def _fin(acc, y, dinv2d, b_obj, b_rel):
    return pl.pallas_call(
        _fin_body,
        out_shape=(jax.ShapeDtypeStruct((N, D), jnp.float32),
                   jax.ShapeDtypeStruct((N, D), jnp.float32)),
    )(acc, y, dinv2d, b_obj, b_rel)


def kernel(obj_vecs, rel_vecs, edge_index, W_obj, b_obj, W_rel, b_rel):
    src = edge_index[:, 0]
    dst = edge_index[:, 1]
    degp = _deg_kernel(dst.reshape(NC, NS, A_CHUNKS, A_B))
    dinv2d = lax.rsqrt(degp[:N] + degp[NP:NP + N] + 1.0)[:, None]
    y = _mm(obj_vecs, rel_vecs, W_obj, W_rel, dinv2d)
    pad = jnp.zeros((E_PAD - E,), jnp.int32)
    src_p = jnp.concatenate([src, pad]).reshape(NS, C_CHUNKS, C_B)
    dst_p = jnp.concatenate([dst, pad - 1]).reshape(NS, C_CHUNKS, C_B)
    acc = _scatter_kernel(y, src_p, dst_p)
    return _fin(acc, y, dinv2d, b_obj, b_rel)


# speedup vs baseline: 13.6094x; 1.1730x over previous
import functools

import jax
import jax.numpy as jnp
from jax import lax
from jax.experimental import pallas as pl
from jax.experimental.pallas import tpu as pltpu
from jax.experimental.pallas import tpu_sc as plsc

N = 10000
D = 128
E = 320000

NC = 2
NS = 16

A_B = 125
A_CHUNKS = E // (NC * NS * A_B)
NP = 10240
NP_TILE = NP // NS


def _deg_body(dst_hbm, out_hbm, idx_v, ones_v, zero_v, buf_v, deg_sh, sem):
    c = lax.axis_index("c")
    s = lax.axis_index("s")
    for j in range(8):
        ones_v[pl.ds(16 * j, 16)] = jnp.full((16,), 1.0, jnp.float32)
        zero_v[pl.ds(16 * j, 16)] = jnp.zeros((16,), jnp.float32)
    for j in range(NP_TILE // 128):
        pltpu.sync_copy(zero_v, deg_sh.at[pl.ds(s * NP_TILE + j * 128, 128)])
    plsc.subcore_barrier()
    pltpu.sync_copy(dst_hbm.at[c, s], idx_v)

    def body(j, carry):
        pltpu.sync_copy(ones_v.at[pl.ds(0, A_B)], deg_sh.at[idx_v.at[j]], add=True)
        return carry

    lax.fori_loop(0, A_CHUNKS, body, 0)
    plsc.subcore_barrier()
    pltpu.sync_copy(deg_sh.at[pl.ds(s * NP_TILE, NP_TILE)], buf_v)
    pltpu.sync_copy(buf_v, out_hbm.at[pl.ds(c * NP + s * NP_TILE, NP_TILE)])


_deg_kernel = functools.partial(
    pl.kernel,
    out_type=jax.ShapeDtypeStruct((NC * NP,), jnp.float32),
    mesh=plsc.VectorSubcoreMesh(core_axis_name="c", subcore_axis_name="s"),
    scratch_types=[
        pltpu.VMEM((A_CHUNKS, A_B), jnp.int32),
        pltpu.VMEM((128,), jnp.float32),
        pltpu.VMEM((128,), jnp.float32),
        pltpu.VMEM((NP_TILE,), jnp.float32),
        pltpu.VMEM_SHARED((NP,), jnp.float32),
        pltpu.SemaphoreType.DMA,
    ],
)(_deg_body)


C_B = 128
C_CHUNKS = 157
E_TILE = C_CHUNKS * C_B
E_PAD = NS * E_TILE
HN = N // 2
ACC_ROWS = HN + 8

def _scatter_body(y_hbm, src_hbm, dst_hbm, out_hbm,
                  srcv, dstv, dlocs, rows, zbuf, acc_sh, sem):
    c = lax.axis_index("c")
    s = lax.axis_index("s")

    def zb(i, carry):
        for j in range(D // 16):
            zbuf[i, pl.ds(16 * j, 16)] = jnp.zeros((16,), jnp.float32)
        return carry

    lax.fori_loop(0, 16, zb, 0)

    def ld(i, carry):
        pltpu.sync_copy(src_hbm.at[s].at[pl.ds(8 * i, 8)],
                        srcv.at[pl.ds(8 * i, 8)])
        pltpu.sync_copy(dst_hbm.at[s].at[pl.ds(8 * i, 8)],
                        dstv.at[pl.ds(8 * i, 8)])
        return carry

    lax.fori_loop(0, C_CHUNKS // 8, ld, 0)
    pltpu.sync_copy(src_hbm.at[s].at[pl.ds(C_CHUNKS - 5, 5)],
                    srcv.at[pl.ds(C_CHUNKS - 5, 5)])
    pltpu.sync_copy(dst_hbm.at[s].at[pl.ds(C_CHUNKS - 5, 5)],
                    dstv.at[pl.ds(C_CHUNKS - 5, 5)])

    for h in range(2):
        lo = h * HN

        def zc(i, carry):
            k = s + 16 * i

            @pl.when(k < ACC_ROWS // 16)
            def _():
                pltpu.sync_copy(zbuf, acc_sh.at[pl.ds(16 * k, 16)])

            return carry

        lax.fori_loop(0, (ACC_ROWS // 16 + NS - 1) // NS, zc, 0)
        plsc.subcore_barrier()

        pltpu.async_copy(y_hbm.at[c].at[srcv.at[0]], rows.at[0], sem)

        def ebody(j, carry):
            b = lax.rem(j, 2)
            for k in range(C_B // 16):
                dv = dstv[j, pl.ds(16 * k, 16)]
                m = (dv >= lo) & (dv < lo + HN)
                dlocs[pl.ds(16 * k, 16)] = jnp.where(
                    m, dv - lo, jnp.int32(HN))
            pltpu.make_async_copy(
                y_hbm.at[c].at[srcv.at[j]], rows.at[b], sem).wait()

            @pl.when(j + 1 < C_CHUNKS)
            def _():
                pltpu.async_copy(
                    y_hbm.at[c].at[srcv.at[j + 1]], rows.at[1 - b], sem)

            pltpu.sync_copy(rows.at[b], acc_sh.at[dlocs], add=True)
            return carry

        lax.fori_loop(0, C_CHUNKS, ebody, 0)
        plsc.subcore_barrier()

        def oc(i, carry):
            k = s + 16 * i

            @pl.when(k < HN // 8)
            def _():
                buf8 = rows.at[0].at[pl.ds(0, 8)]
                pltpu.sync_copy(acc_sh.at[pl.ds(8 * k, 8)], buf8)
                pltpu.sync_copy(buf8, out_hbm.at[c].at[pl.ds(lo + 8 * k, 8)])

            return carry

        lax.fori_loop(0, (HN // 8 + NS - 1) // NS, oc, 0)
        if h == 0:
            plsc.subcore_barrier()


_scatter_kernel = functools.partial(
    pl.kernel,
    out_type=jax.ShapeDtypeStruct((NC, N, D), jnp.float32),
    mesh=plsc.VectorSubcoreMesh(core_axis_name="c", subcore_axis_name="s"),
    scratch_types=[
        pltpu.VMEM((C_CHUNKS, C_B), jnp.int32),
        pltpu.VMEM((C_CHUNKS, C_B), jnp.int32),
        pltpu.VMEM((C_B,), jnp.int32),
        pltpu.VMEM((2, C_B, D), jnp.float32),
        pltpu.VMEM((16, D), jnp.float32),
        pltpu.VMEM_SHARED((ACC_ROWS, D), jnp.float32),
        pltpu.SemaphoreType.DMA,
    ],
)(_scatter_body)


def _mm_body(obj_ref, rel_ref, wo_ref, wr_ref, dinv_ref, y_ref):
    dinv = dinv_ref[...]
    y_ref[0] = lax.dot_general(obj_ref[...], wo_ref[...],
                               (((1,), (1,)), ((), ())),
                               preferred_element_type=jnp.float32) * dinv
    y_ref[1] = lax.dot_general(rel_ref[...], wr_ref[...],
                               (((1,), (1,)), ((), ())),
                               preferred_element_type=jnp.float32) * dinv


def _mm(obj_vecs, rel_vecs, W_obj, W_rel, dinv2d):
    return pl.pallas_call(
        _mm_body,
        out_shape=jax.ShapeDtypeStruct((NC, N, D), jnp.float32),
    )(obj_vecs, rel_vecs, W_obj, W_rel, dinv2d)


def _fin_body(acc_ref, y_ref, dinv_ref, bo_ref, br_ref, oo_ref, or_ref):
    dinv = dinv_ref[...]
    oo_ref[...] = jnp.maximum(
        (acc_ref[0] + y_ref[0]) * dinv + bo_ref[...][None, :], 0.0)
    or_ref[...] = jnp.maximum(
        (acc_ref[1] + y_ref[1]) * dinv + br_ref[...][None, :], 0.0)


def _fin(acc, y, dinv2d, b_obj, b_rel):
    return pl.pallas_call(
        _fin_body,
        out_shape=(jax.ShapeDtypeStruct((N, D), jnp.float32),
                   jax.ShapeDtypeStruct((N, D), jnp.float32)),
    )(acc, y, dinv2d, b_obj, b_rel)


def kernel(obj_vecs, rel_vecs, edge_index, W_obj, b_obj, W_rel, b_rel):
    src = edge_index[:, 0]
    dst = edge_index[:, 1]
    degp = _deg_kernel(dst.reshape(NC, NS, A_CHUNKS, A_B))
    dinv2d = lax.rsqrt(degp[:N] + degp[NP:NP + N] + 1.0)[:, None]
    y = _mm(obj_vecs, rel_vecs, W_obj, W_rel, dinv2d)
    pad = jnp.zeros((E_PAD - E,), jnp.int32)
    src_p = jnp.concatenate([src, pad]).reshape(NS, C_CHUNKS, C_B)
    dst_p = jnp.concatenate([dst, pad - 1]).reshape(NS, C_CHUNKS, C_B)
    acc = _scatter_kernel(y, src_p, dst_p)
    return _fin(acc, y, dinv2d, b_obj, b_rel)


# speedup vs baseline: 24.9442x; 1.8329x over previous
import functools

import jax
import jax.numpy as jnp
from jax import lax
from jax.experimental import pallas as pl
from jax.experimental.pallas import tpu as pltpu
from jax.experimental.pallas import tpu_sc as plsc

N = 10000
D = 128
E = 320000

NC = 2
NS = 16

A_B = 125
A_CHUNKS = E // (NC * NS * A_B)
NP = 10240
NP_TILE = NP // NS


def _deg_body(dst_hbm, out_hbm, idx_v, ones_v, zero_v, buf_v, deg_sh, sem):
    c = lax.axis_index("c")
    s = lax.axis_index("s")
    for j in range(8):
        ones_v[pl.ds(16 * j, 16)] = jnp.full((16,), 1.0, jnp.float32)
        zero_v[pl.ds(16 * j, 16)] = jnp.zeros((16,), jnp.float32)
    for j in range(NP_TILE // 128):
        pltpu.sync_copy(zero_v, deg_sh.at[pl.ds(s * NP_TILE + j * 128, 128)])
    plsc.subcore_barrier()
    pltpu.sync_copy(dst_hbm.at[c, s], idx_v)

    def body(j, carry):
        pltpu.sync_copy(ones_v.at[pl.ds(0, A_B)], deg_sh.at[idx_v.at[j]], add=True)
        return carry

    lax.fori_loop(0, A_CHUNKS, body, 0)
    plsc.subcore_barrier()
    pltpu.sync_copy(deg_sh.at[pl.ds(s * NP_TILE, NP_TILE)], buf_v)
    pltpu.sync_copy(buf_v, out_hbm.at[pl.ds(c * NP + s * NP_TILE, NP_TILE)])


_deg_kernel = functools.partial(
    pl.kernel,
    out_type=jax.ShapeDtypeStruct((NC * NP,), jnp.float32),
    mesh=plsc.VectorSubcoreMesh(core_axis_name="c", subcore_axis_name="s"),
    scratch_types=[
        pltpu.VMEM((A_CHUNKS, A_B), jnp.int32),
        pltpu.VMEM((128,), jnp.float32),
        pltpu.VMEM((128,), jnp.float32),
        pltpu.VMEM((NP_TILE,), jnp.float32),
        pltpu.VMEM_SHARED((NP,), jnp.float32),
        pltpu.SemaphoreType.DMA,
    ],
)(_deg_body)


C_B = 128
C_CHUNKS = 157
E_TILE = C_CHUNKS * C_B
E_PAD = NS * E_TILE
ACC_ROWS = N + 16
PK_ROWS = 80


def _scatter_body(y_hbm, pk_hbm, out_hbm,
                  pkv, srcw, dlocs, rows, acc_sh, sem):
    c = lax.axis_index("c")
    s = lax.axis_index("s")
    zrows = rows.at[0]

    def zb(i, carry):
        for j in range(D // 16):
            zrows[i, pl.ds(16 * j, 16)] = jnp.zeros((16,), jnp.float32)
        return carry

    lax.fori_loop(0, 16, zb, 0)

    def zc(i, carry):
        k = s + 16 * i

        @pl.when(k < ACC_ROWS // 16)
        def _():
            pltpu.sync_copy(zrows.at[pl.ds(0, 16)], acc_sh.at[pl.ds(16 * k, 16)])

        return carry

    lax.fori_loop(0, (ACC_ROWS // 16 + NS - 1) // NS, zc, 0)
    pltpu.sync_copy(pk_hbm.at[s].at[pl.ds(0, PK_ROWS)], pkv)
    plsc.subcore_barrier()

    def widen_src(jj, bb):
        for k in range(C_B // 16):
            v = pkv[jj, pl.ds(16 * k, 16)]
            srcw[bb, pl.ds(16 * k, 16)] = v & jnp.int32(0xFFFF)

    def widen_dst(jj):
        for k in range(C_B // 16):
            v = pkv[jj, pl.ds(16 * k, 16)]
            d = lax.shift_right_logical(v, 16)
            dlocs[pl.ds(16 * k, 16)] = jnp.where(d < N, d, jnp.int32(N))

    widen_src(0, 0)
    pltpu.async_copy(y_hbm.at[c].at[srcw.at[0]], rows.at[0], sem)

    def ebody(j, carry):
        b = lax.rem(j, 2)
        widen_dst(lax.rem(j, PK_ROWS))

        @pl.when(j == PK_ROWS - 1)
        def _():
            pltpu.sync_copy(
                pk_hbm.at[s].at[pl.ds(PK_ROWS, C_CHUNKS - PK_ROWS)],
                pkv.at[pl.ds(0, C_CHUNKS - PK_ROWS)])

        @pl.when(j + 1 < C_CHUNKS)
        def _():
            widen_src(lax.rem(j + 1, PK_ROWS), 1 - b)

        pltpu.make_async_copy(
            y_hbm.at[c].at[srcw.at[b]], rows.at[b], sem).wait()

        @pl.when(j + 1 < C_CHUNKS)
        def _():
            pltpu.async_copy(
                y_hbm.at[c].at[srcw.at[1 - b]], rows.at[1 - b], sem)

        pltpu.sync_copy(rows.at[b], acc_sh.at[dlocs], add=True)
        return carry

    lax.fori_loop(0, C_CHUNKS, ebody, 0)
    plsc.subcore_barrier()

    def oc(i, carry):
        k = s + 16 * i

        @pl.when(k < N // 8)
        def _():
            buf8 = rows.at[0].at[pl.ds(0, 8)]
            pltpu.sync_copy(acc_sh.at[pl.ds(8 * k, 8)], buf8)
            pltpu.sync_copy(buf8, out_hbm.at[c].at[pl.ds(8 * k, 8)])

        return carry

    lax.fori_loop(0, (N // 8 + NS - 1) // NS, oc, 0)


_scatter_kernel = functools.partial(
    pl.kernel,
    out_type=jax.ShapeDtypeStruct((NC, N, D), jnp.float32),
    mesh=plsc.VectorSubcoreMesh(core_axis_name="c", subcore_axis_name="s"),
    scratch_types=[
        pltpu.VMEM((PK_ROWS, C_B), jnp.int32),
        pltpu.VMEM((2, C_B), jnp.int32),
        pltpu.VMEM((C_B,), jnp.int32),
        pltpu.VMEM((2, C_B, D), jnp.float32),
        pltpu.VMEM_SHARED((ACC_ROWS, D), jnp.float32),
        pltpu.SemaphoreType.DMA,
    ],
)(_scatter_body)


def _mm_body(obj_ref, rel_ref, wo_ref, wr_ref, dinv_ref, y_ref):
    dinv = dinv_ref[...]
    y_ref[0] = lax.dot_general(obj_ref[...], wo_ref[...],
                               (((1,), (1,)), ((), ())),
                               preferred_element_type=jnp.float32) * dinv
    y_ref[1] = lax.dot_general(rel_ref[...], wr_ref[...],
                               (((1,), (1,)), ((), ())),
                               preferred_element_type=jnp.float32) * dinv


def _mm(obj_vecs, rel_vecs, W_obj, W_rel, dinv2d):
    return pl.pallas_call(
        _mm_body,
        out_shape=jax.ShapeDtypeStruct((NC, N, D), jnp.float32),
    )(obj_vecs, rel_vecs, W_obj, W_rel, dinv2d)


def _fin_body(acc_ref, y_ref, dinv_ref, bo_ref, br_ref, oo_ref, or_ref):
    dinv = dinv_ref[...]
    oo_ref[...] = jnp.maximum(
        (acc_ref[0] + y_ref[0]) * dinv + bo_ref[...][None, :], 0.0)
    or_ref[...] = jnp.maximum(
        (acc_ref[1] + y_ref[1]) * dinv + br_ref[...][None, :], 0.0)


def _fin(acc, y, dinv2d, b_obj, b_rel):
    return pl.pallas_call(
        _fin_body,
        out_shape=(jax.ShapeDtypeStruct((N, D), jnp.float32),
                   jax.ShapeDtypeStruct((N, D), jnp.float32)),
    )(acc, y, dinv2d, b_obj, b_rel)


def kernel(obj_vecs, rel_vecs, edge_index, W_obj, b_obj, W_rel, b_rel):
    src = edge_index[:, 0]
    dst = edge_index[:, 1]
    degp = _deg_kernel(dst.reshape(NC, NS, A_CHUNKS, A_B))
    dinv2d = lax.rsqrt(degp[:N] + degp[NP:NP + N] + 1.0)[:, None]
    y = _mm(obj_vecs, rel_vecs, W_obj, W_rel, dinv2d)
    pk = jnp.left_shift(dst, 16) | src
    pad = jnp.full((E_PAD - E,), jnp.int32(0xFFFF) << 16, jnp.int32)
    pk_p = jnp.concatenate([pk, pad]).reshape(NS, C_CHUNKS, C_B)
    acc = _scatter_kernel(y, pk_p)
    return _fin(acc, y, dinv2d, b_obj, b_rel)


# speedup vs baseline: 25.7821x; 1.0336x over previous
import functools

import jax
import jax.numpy as jnp
from jax import lax
from jax.experimental import pallas as pl
from jax.experimental.pallas import tpu as pltpu
from jax.experimental.pallas import tpu_sc as plsc

N = 10000
D = 128
E = 320000

NC = 2
NS = 16

A_B = 125
A_CHUNKS = E // (NC * NS * A_B)
NP = 10240
NP_TILE = NP // NS


def _deg_body(dst_hbm, out_hbm, idx_v, ones_v, zero_v, buf_v, deg_sh, sem):
    c = lax.axis_index("c")
    s = lax.axis_index("s")
    for j in range(8):
        ones_v[pl.ds(16 * j, 16)] = jnp.full((16,), 1.0, jnp.float32)
        zero_v[pl.ds(16 * j, 16)] = jnp.zeros((16,), jnp.float32)
    for j in range(NP_TILE // 128):
        pltpu.sync_copy(zero_v, deg_sh.at[pl.ds(s * NP_TILE + j * 128, 128)])
    plsc.subcore_barrier()
    pltpu.sync_copy(dst_hbm.at[c, s], idx_v)

    def body(j, carry):
        pltpu.sync_copy(ones_v.at[pl.ds(0, A_B)], deg_sh.at[idx_v.at[j]], add=True)
        return carry

    lax.fori_loop(0, A_CHUNKS, body, 0)
    plsc.subcore_barrier()
    pltpu.sync_copy(deg_sh.at[pl.ds(s * NP_TILE, NP_TILE)], buf_v)
    pltpu.sync_copy(buf_v, out_hbm.at[pl.ds(c * NP + s * NP_TILE, NP_TILE)])


_deg_kernel = functools.partial(
    pl.kernel,
    out_type=jax.ShapeDtypeStruct((NC * NP,), jnp.float32),
    mesh=plsc.VectorSubcoreMesh(core_axis_name="c", subcore_axis_name="s"),
    scratch_types=[
        pltpu.VMEM((A_CHUNKS, A_B), jnp.int32),
        pltpu.VMEM((128,), jnp.float32),
        pltpu.VMEM((128,), jnp.float32),
        pltpu.VMEM((NP_TILE,), jnp.float32),
        pltpu.VMEM_SHARED((NP,), jnp.float32),
        pltpu.SemaphoreType.DMA,
    ],
)(_deg_body)


C_B = 128
C_CHUNKS = 157
E_TILE = C_CHUNKS * C_B
E_PAD = NS * E_TILE
Z_ROWS = 80
ACC_ROWS = 10080
PK_ROWS = 80


def _scatter_body(y_hbm, pk_hbm, out_hbm,
                  pkv, srcw, dlocs, rows, acc_sh, sem, sem2):
    c = lax.axis_index("c")
    s = lax.axis_index("s")
    zrows = rows.at[0]

    def zb(i, carry):
        for j in range(D // 16):
            zrows[i, pl.ds(16 * j, 16)] = jnp.zeros((16,), jnp.float32)
        return carry

    lax.fori_loop(0, Z_ROWS, zb, 0)

    def zc(i, carry):
        k = s + 16 * i

        @pl.when(k < ACC_ROWS // Z_ROWS)
        def _():
            pltpu.sync_copy(zrows.at[pl.ds(0, Z_ROWS)],
                            acc_sh.at[pl.ds(Z_ROWS * k, Z_ROWS)])

        return carry

    lax.fori_loop(0, (ACC_ROWS // Z_ROWS + NS - 1) // NS, zc, 0)
    pltpu.sync_copy(pk_hbm.at[s].at[pl.ds(0, PK_ROWS)], pkv)
    plsc.subcore_barrier()

    def widen_src(jj, bb):
        for k in range(C_B // 16):
            v = pkv[jj, pl.ds(16 * k, 16)]
            srcw[bb, pl.ds(16 * k, 16)] = v & jnp.int32(0xFFFF)

    def widen_dst(jj):
        for k in range(C_B // 16):
            v = pkv[jj, pl.ds(16 * k, 16)]
            d = lax.shift_right_logical(v, 16)
            dlocs[pl.ds(16 * k, 16)] = jnp.where(d < N, d, jnp.int32(N))

    widen_src(0, 0)
    pltpu.async_copy(y_hbm.at[c].at[srcw.at[0]], rows.at[0], sem)

    def ebody(j, carry):
        b = lax.rem(j, 2)
        widen_dst(lax.rem(j, PK_ROWS))

        @pl.when(j == PK_ROWS - 3)
        def _():
            pltpu.async_copy(
                pk_hbm.at[s].at[pl.ds(PK_ROWS, C_CHUNKS - PK_ROWS)],
                pkv.at[pl.ds(0, C_CHUNKS - PK_ROWS)], sem2)

        @pl.when(j == PK_ROWS - 1)
        def _():
            pltpu.make_async_copy(
                pk_hbm.at[s].at[pl.ds(PK_ROWS, C_CHUNKS - PK_ROWS)],
                pkv.at[pl.ds(0, C_CHUNKS - PK_ROWS)], sem2).wait()

        @pl.when(j + 1 < C_CHUNKS)
        def _():
            widen_src(lax.rem(j + 1, PK_ROWS), 1 - b)

        pltpu.make_async_copy(
            y_hbm.at[c].at[srcw.at[b]], rows.at[b], sem).wait()

        @pl.when(j + 1 < C_CHUNKS)
        def _():
            pltpu.async_copy(
                y_hbm.at[c].at[srcw.at[1 - b]], rows.at[1 - b], sem)

        pltpu.sync_copy(rows.at[b], acc_sh.at[dlocs], add=True)
        return carry

    lax.fori_loop(0, C_CHUNKS, ebody, 0)
    plsc.subcore_barrier()

    def oc(i, carry):
        k = s + 16 * i

        @pl.when(k < N // Z_ROWS)
        def _():
            pltpu.sync_copy(acc_sh.at[pl.ds(Z_ROWS * k, Z_ROWS)],
                            out_hbm.at[c].at[pl.ds(Z_ROWS * k, Z_ROWS)])

        return carry

    lax.fori_loop(0, (N // Z_ROWS + NS - 1) // NS, oc, 0)


_scatter_kernel = functools.partial(
    pl.kernel,
    out_type=jax.ShapeDtypeStruct((NC, N, D), jnp.float32),
    mesh=plsc.VectorSubcoreMesh(core_axis_name="c", subcore_axis_name="s"),
    scratch_types=[
        pltpu.VMEM((PK_ROWS, C_B), jnp.int32),
        pltpu.VMEM((2, C_B), jnp.int32),
        pltpu.VMEM((C_B,), jnp.int32),
        pltpu.VMEM((2, C_B, D), jnp.float32),
        pltpu.VMEM_SHARED((ACC_ROWS, D), jnp.float32),
        pltpu.SemaphoreType.DMA,
        pltpu.SemaphoreType.DMA,
    ],
)(_scatter_body)


def _mm_body(obj_ref, rel_ref, wo_ref, wr_ref, dinv_ref, y_ref):
    dinv = dinv_ref[...]
    y_ref[0] = lax.dot_general(obj_ref[...], wo_ref[...],
                               (((1,), (1,)), ((), ())),
                               preferred_element_type=jnp.float32) * dinv
    y_ref[1] = lax.dot_general(rel_ref[...], wr_ref[...],
                               (((1,), (1,)), ((), ())),
                               preferred_element_type=jnp.float32) * dinv


def _mm(obj_vecs, rel_vecs, W_obj, W_rel, dinv2d):
    return pl.pallas_call(
        _mm_body,
        out_shape=jax.ShapeDtypeStruct((NC, N, D), jnp.float32),
    )(obj_vecs, rel_vecs, W_obj, W_rel, dinv2d)


def _fin_body(acc_ref, y_ref, dinv_ref, bo_ref, br_ref, oo_ref, or_ref):
    dinv = dinv_ref[...]
    oo_ref[...] = jnp.maximum(
        (acc_ref[0] + y_ref[0]) * dinv + bo_ref[...][None, :], 0.0)
    or_ref[...] = jnp.maximum(
        (acc_ref[1] + y_ref[1]) * dinv + br_ref[...][None, :], 0.0)


def _fin(acc, y, dinv2d, b_obj, b_rel):
    return pl.pallas_call(
        _fin_body,
        out_shape=(jax.ShapeDtypeStruct((N, D), jnp.float32),
                   jax.ShapeDtypeStruct((N, D), jnp.float32)),
    )(acc, y, dinv2d, b_obj, b_rel)


def kernel(obj_vecs, rel_vecs, edge_index, W_obj, b_obj, W_rel, b_rel):
    src = edge_index[:, 0]
    dst = edge_index[:, 1]
    degp = _deg_kernel(dst.reshape(NC, NS, A_CHUNKS, A_B))
    dinv2d = lax.rsqrt(degp[:N] + degp[NP:NP + N] + 1.0)[:, None]
    y = _mm(obj_vecs, rel_vecs, W_obj, W_rel, dinv2d)
    pk = jnp.left_shift(dst, 16) | src
    pad = jnp.full((E_PAD - E,), jnp.int32(0xFFFF) << 16, jnp.int32)
    pk_p = jnp.concatenate([pk, pad]).reshape(NS, C_CHUNKS, C_B)
    acc = _scatter_kernel(y, pk_p)
    return _fin(acc, y, dinv2d, b_obj, b_rel)


# speedup vs baseline: 26.3354x; 1.0215x over previous
import functools

import jax
import jax.numpy as jnp
from jax import lax
from jax.experimental import pallas as pl
from jax.experimental.pallas import tpu as pltpu
from jax.experimental.pallas import tpu_sc as plsc

N = 10000
D = 128
E = 320000

NC = 2
NS = 16

A_B = 125
A_CHUNKS = E // (NC * NS * A_B)
NP = 10240
NP_TILE = NP // NS


def _deg_body(dst_hbm, out_hbm, idx_v, ones_v, zero_v, buf_v, deg_sh, sem):
    c = lax.axis_index("c")
    s = lax.axis_index("s")
    for j in range(8):
        ones_v[pl.ds(16 * j, 16)] = jnp.full((16,), 1.0, jnp.float32)
        zero_v[pl.ds(16 * j, 16)] = jnp.zeros((16,), jnp.float32)
    for j in range(NP_TILE // 128):
        pltpu.sync_copy(zero_v, deg_sh.at[pl.ds(s * NP_TILE + j * 128, 128)])
    plsc.subcore_barrier()
    pltpu.sync_copy(dst_hbm.at[c, s], idx_v)

    def body(j, carry):
        pltpu.async_copy(ones_v.at[pl.ds(0, A_B)], deg_sh.at[idx_v.at[j]],
                         sem, add=True)
        return carry

    lax.fori_loop(0, A_CHUNKS, body, 0)

    def bodyw(j, carry):
        pltpu.make_async_copy(ones_v.at[pl.ds(0, A_B)], deg_sh.at[idx_v.at[j]],
                              sem).wait()
        return carry

    lax.fori_loop(0, A_CHUNKS, bodyw, 0)
    plsc.subcore_barrier()
    pltpu.sync_copy(deg_sh.at[pl.ds(s * NP_TILE, NP_TILE)], buf_v)
    pltpu.sync_copy(buf_v, out_hbm.at[pl.ds(c * NP + s * NP_TILE, NP_TILE)])


_deg_kernel = functools.partial(
    pl.kernel,
    out_type=jax.ShapeDtypeStruct((NC * NP,), jnp.float32),
    mesh=plsc.VectorSubcoreMesh(core_axis_name="c", subcore_axis_name="s"),
    scratch_types=[
        pltpu.VMEM((A_CHUNKS, A_B), jnp.int32),
        pltpu.VMEM((128,), jnp.float32),
        pltpu.VMEM((128,), jnp.float32),
        pltpu.VMEM((NP_TILE,), jnp.float32),
        pltpu.VMEM_SHARED((NP,), jnp.float32),
        pltpu.SemaphoreType.DMA,
    ],
)(_deg_body)


C_B = 128
C_CHUNKS = 157
E_TILE = C_CHUNKS * C_B
E_PAD = NS * E_TILE
Z_ROWS = 80
ACC_ROWS = 10080
PK_ROWS = 80


def _scatter_body(y_hbm, pk_hbm, out_hbm,
                  pkv, srcw, dlocs, rows, acc_sh, sem, sem2):
    c = lax.axis_index("c")
    s = lax.axis_index("s")
    zrows = rows.at[0]

    def zb(i, carry):
        for j in range(D // 16):
            zrows[i, pl.ds(16 * j, 16)] = jnp.zeros((16,), jnp.float32)
        return carry

    lax.fori_loop(0, Z_ROWS, zb, 0)

    def zc(i, carry):
        k = s + 16 * i

        @pl.when(k < ACC_ROWS // Z_ROWS)
        def _():
            pltpu.sync_copy(zrows.at[pl.ds(0, Z_ROWS)],
                            acc_sh.at[pl.ds(Z_ROWS * k, Z_ROWS)])

        return carry

    lax.fori_loop(0, (ACC_ROWS // Z_ROWS + NS - 1) // NS, zc, 0)
    pltpu.sync_copy(pk_hbm.at[s].at[pl.ds(0, PK_ROWS)], pkv)
    plsc.subcore_barrier()

    def widen_src(jj, bb):
        for k in range(C_B // 16):
            v = pkv[jj, pl.ds(16 * k, 16)]
            srcw[bb, pl.ds(16 * k, 16)] = v & jnp.int32(0xFFFF)

    def widen_dst(jj):
        for k in range(C_B // 16):
            v = pkv[jj, pl.ds(16 * k, 16)]
            d = lax.shift_right_logical(v, 16)
            dlocs[pl.ds(16 * k, 16)] = jnp.where(d < N, d, jnp.int32(N))

    widen_src(0, 0)
    pltpu.async_copy(y_hbm.at[c].at[srcw.at[0]], rows.at[0], sem)

    def ebody(j, carry):
        b = lax.rem(j, 2)
        widen_dst(lax.rem(j, PK_ROWS))

        @pl.when(j == PK_ROWS - 3)
        def _():
            pltpu.async_copy(
                pk_hbm.at[s].at[pl.ds(PK_ROWS, C_CHUNKS - PK_ROWS)],
                pkv.at[pl.ds(0, C_CHUNKS - PK_ROWS)], sem2)

        @pl.when(j == PK_ROWS - 1)
        def _():
            pltpu.make_async_copy(
                pk_hbm.at[s].at[pl.ds(PK_ROWS, C_CHUNKS - PK_ROWS)],
                pkv.at[pl.ds(0, C_CHUNKS - PK_ROWS)], sem2).wait()

        @pl.when(j + 1 < C_CHUNKS)
        def _():
            widen_src(lax.rem(j + 1, PK_ROWS), 1 - b)

        pltpu.make_async_copy(
            y_hbm.at[c].at[srcw.at[b]], rows.at[b], sem).wait()

        @pl.when(j + 1 < C_CHUNKS)
        def _():
            pltpu.async_copy(
                y_hbm.at[c].at[srcw.at[1 - b]], rows.at[1 - b], sem)

        pltpu.sync_copy(rows.at[b], acc_sh.at[dlocs], add=True)
        return carry

    lax.fori_loop(0, C_CHUNKS, ebody, 0)
    plsc.subcore_barrier()

    def oc(i, carry):
        k = s + 16 * i

        @pl.when(k < N // Z_ROWS)
        def _():
            pltpu.sync_copy(acc_sh.at[pl.ds(Z_ROWS * k, Z_ROWS)],
                            out_hbm.at[c].at[pl.ds(Z_ROWS * k, Z_ROWS)])

        return carry

    lax.fori_loop(0, (N // Z_ROWS + NS - 1) // NS, oc, 0)


_scatter_kernel = functools.partial(
    pl.kernel,
    out_type=jax.ShapeDtypeStruct((NC, N, D), jnp.float32),
    mesh=plsc.VectorSubcoreMesh(core_axis_name="c", subcore_axis_name="s"),
    scratch_types=[
        pltpu.VMEM((PK_ROWS, C_B), jnp.int32),
        pltpu.VMEM((2, C_B), jnp.int32),
        pltpu.VMEM((C_B,), jnp.int32),
        pltpu.VMEM((2, C_B, D), jnp.float32),
        pltpu.VMEM_SHARED((ACC_ROWS, D), jnp.float32),
        pltpu.SemaphoreType.DMA,
        pltpu.SemaphoreType.DMA,
    ],
)(_scatter_body)


def _mm_body(obj_ref, rel_ref, wo_ref, wr_ref, dinv_ref, y_ref):
    dinv = dinv_ref[...]
    y_ref[0] = lax.dot_general(obj_ref[...], wo_ref[...],
                               (((1,), (1,)), ((), ())),
                               preferred_element_type=jnp.float32) * dinv
    y_ref[1] = lax.dot_general(rel_ref[...], wr_ref[...],
                               (((1,), (1,)), ((), ())),
                               preferred_element_type=jnp.float32) * dinv


def _mm(obj_vecs, rel_vecs, W_obj, W_rel, dinv2d):
    return pl.pallas_call(
        _mm_body,
        out_shape=jax.ShapeDtypeStruct((NC, N, D), jnp.float32),
    )(obj_vecs, rel_vecs, W_obj, W_rel, dinv2d)


def _fin_body(acc_ref, y_ref, dinv_ref, bo_ref, br_ref, oo_ref, or_ref):
    dinv = dinv_ref[...]
    oo_ref[...] = jnp.maximum(
        (acc_ref[0] + y_ref[0]) * dinv + bo_ref[...][None, :], 0.0)
    or_ref[...] = jnp.maximum(
        (acc_ref[1] + y_ref[1]) * dinv + br_ref[...][None, :], 0.0)


def _fin(acc, y, dinv2d, b_obj, b_rel):
    return pl.pallas_call(
        _fin_body,
        out_shape=(jax.ShapeDtypeStruct((N, D), jnp.float32),
                   jax.ShapeDtypeStruct((N, D), jnp.float32)),
    )(acc, y, dinv2d, b_obj, b_rel)


def kernel(obj_vecs, rel_vecs, edge_index, W_obj, b_obj, W_rel, b_rel):
    src = edge_index[:, 0]
    dst = edge_index[:, 1]
    degp = _deg_kernel(dst.reshape(NC, NS, A_CHUNKS, A_B))
    dinv2d = lax.rsqrt(degp[:N] + degp[NP:NP + N] + 1.0)[:, None]
    y = _mm(obj_vecs, rel_vecs, W_obj, W_rel, dinv2d)
    pk = jnp.left_shift(dst, 16) | src
    pad = jnp.full((E_PAD - E,), jnp.int32(0xFFFF) << 16, jnp.int32)
    pk_p = jnp.concatenate([pk, pad]).reshape(NS, C_CHUNKS, C_B)
    acc = _scatter_kernel(y, pk_p)
    return _fin(acc, y, dinv2d, b_obj, b_rel)


# speedup vs baseline: 28.2517x; 1.0728x over previous
import functools

import jax
import jax.numpy as jnp
from jax import lax
from jax.experimental import pallas as pl
from jax.experimental.pallas import tpu as pltpu
from jax.experimental.pallas import tpu_sc as plsc

N = 10000
D = 128
E = 320000

NC = 2
NS = 16

A_B = 125
A_CHUNKS = E // (NC * NS * A_B)
NP = 10240
NP_TILE = NP // NS


def _deg_body(dst_hbm, out_hbm, idx_v, ones_v, zero_v, buf_v, deg_sh, sem):
    c = lax.axis_index("c")
    s = lax.axis_index("s")
    for j in range(8):
        ones_v[pl.ds(16 * j, 16)] = jnp.full((16,), 1.0, jnp.float32)
        zero_v[pl.ds(16 * j, 16)] = jnp.zeros((16,), jnp.float32)
    pltpu.async_copy(dst_hbm.at[c, s], idx_v, sem)
    for j in range(NP_TILE // 128):
        pltpu.sync_copy(zero_v, deg_sh.at[pl.ds(s * NP_TILE + j * 128, 128)])
    plsc.subcore_barrier()
    pltpu.make_async_copy(dst_hbm.at[c, s], idx_v, sem).wait()

    def body(j, carry):
        pltpu.async_copy(ones_v.at[pl.ds(0, A_B)], deg_sh.at[idx_v.at[j]],
                         sem, add=True)
        return carry

    lax.fori_loop(0, A_CHUNKS, body, 0)

    def bodyw(j, carry):
        pltpu.make_async_copy(ones_v.at[pl.ds(0, A_B)], deg_sh.at[idx_v.at[j]],
                              sem).wait()
        return carry

    lax.fori_loop(0, A_CHUNKS, bodyw, 0)
    plsc.subcore_barrier()
    pltpu.sync_copy(deg_sh.at[pl.ds(s * NP_TILE, NP_TILE)], buf_v)
    pltpu.sync_copy(buf_v, out_hbm.at[pl.ds(c * NP + s * NP_TILE, NP_TILE)])


_deg_kernel = functools.partial(
    pl.kernel,
    out_type=jax.ShapeDtypeStruct((NC * NP,), jnp.float32),
    mesh=plsc.VectorSubcoreMesh(core_axis_name="c", subcore_axis_name="s"),
    scratch_types=[
        pltpu.VMEM((A_CHUNKS, A_B), jnp.int32),
        pltpu.VMEM((128,), jnp.float32),
        pltpu.VMEM((128,), jnp.float32),
        pltpu.VMEM((NP_TILE,), jnp.float32),
        pltpu.VMEM_SHARED((NP,), jnp.float32),
        pltpu.SemaphoreType.DMA,
    ],
)(_deg_body)


C_B = 128
C_CHUNKS = 157
E_TILE = C_CHUNKS * C_B
E_PAD = NS * E_TILE
Z_ROWS = 80
ACC_ROWS = 10080
PK_ROWS = 80


def _scatter_body(y_hbm, pk_hbm, out_hbm,
                  pkv, srcw, dlocs, rows, acc_sh, sem, sem2):
    c = lax.axis_index("c")
    s = lax.axis_index("s")
    zrows = rows.at[0]

    def zb(i, carry):
        for j in range(D // 16):
            zrows[i, pl.ds(16 * j, 16)] = jnp.zeros((16,), jnp.float32)
        return carry

    lax.fori_loop(0, Z_ROWS, zb, 0)

    def zc(i, carry):
        k = s + 16 * i

        @pl.when(k < ACC_ROWS // Z_ROWS)
        def _():
            pltpu.sync_copy(zrows.at[pl.ds(0, Z_ROWS)],
                            acc_sh.at[pl.ds(Z_ROWS * k, Z_ROWS)])

        return carry

    pltpu.async_copy(pk_hbm.at[s].at[pl.ds(0, PK_ROWS)], pkv, sem2)
    lax.fori_loop(0, (ACC_ROWS // Z_ROWS + NS - 1) // NS, zc, 0)
    pltpu.make_async_copy(pk_hbm.at[s].at[pl.ds(0, PK_ROWS)], pkv, sem2).wait()
    plsc.subcore_barrier()

    def widen_src(jj, bb):
        for k in range(C_B // 16):
            v = pkv[jj, pl.ds(16 * k, 16)]
            srcw[bb, pl.ds(16 * k, 16)] = v & jnp.int32(0xFFFF)

    def widen_dst(jj):
        for k in range(C_B // 16):
            v = pkv[jj, pl.ds(16 * k, 16)]
            d = lax.shift_right_logical(v, 16)
            dlocs[pl.ds(16 * k, 16)] = jnp.where(d < N, d, jnp.int32(N))

    widen_src(0, 0)
    pltpu.async_copy(y_hbm.at[c].at[srcw.at[0]], rows.at[0], sem)

    def ebody(j, carry):
        b = lax.rem(j, 2)
        widen_dst(lax.rem(j, PK_ROWS))

        @pl.when(j == PK_ROWS - 3)
        def _():
            pltpu.async_copy(
                pk_hbm.at[s].at[pl.ds(PK_ROWS, C_CHUNKS - PK_ROWS)],
                pkv.at[pl.ds(0, C_CHUNKS - PK_ROWS)], sem2)

        @pl.when(j == PK_ROWS - 1)
        def _():
            pltpu.make_async_copy(
                pk_hbm.at[s].at[pl.ds(PK_ROWS, C_CHUNKS - PK_ROWS)],
                pkv.at[pl.ds(0, C_CHUNKS - PK_ROWS)], sem2).wait()

        @pl.when(j + 1 < C_CHUNKS)
        def _():
            widen_src(lax.rem(j + 1, PK_ROWS), 1 - b)

        pltpu.make_async_copy(
            y_hbm.at[c].at[srcw.at[b]], rows.at[b], sem).wait()

        @pl.when(j + 1 < C_CHUNKS)
        def _():
            pltpu.async_copy(
                y_hbm.at[c].at[srcw.at[1 - b]], rows.at[1 - b], sem)

        pltpu.sync_copy(rows.at[b], acc_sh.at[dlocs], add=True)
        return carry

    lax.fori_loop(0, C_CHUNKS, ebody, 0)
    plsc.subcore_barrier()

    def oc(i, carry):
        k = s + 16 * i

        @pl.when(k < N // Z_ROWS)
        def _():
            pltpu.sync_copy(acc_sh.at[pl.ds(Z_ROWS * k, Z_ROWS)],
                            out_hbm.at[c].at[pl.ds(Z_ROWS * k, Z_ROWS)])

        return carry

    lax.fori_loop(0, (N // Z_ROWS + NS - 1) // NS, oc, 0)


_scatter_kernel = functools.partial(
    pl.kernel,
    out_type=jax.ShapeDtypeStruct((NC, N, D), jnp.float32),
    mesh=plsc.VectorSubcoreMesh(core_axis_name="c", subcore_axis_name="s"),
    scratch_types=[
        pltpu.VMEM((PK_ROWS, C_B), jnp.int32),
        pltpu.VMEM((2, C_B), jnp.int32),
        pltpu.VMEM((C_B,), jnp.int32),
        pltpu.VMEM((2, C_B, D), jnp.float32),
        pltpu.VMEM_SHARED((ACC_ROWS, D), jnp.float32),
        pltpu.SemaphoreType.DMA,
        pltpu.SemaphoreType.DMA,
    ],
)(_scatter_body)


def _dinv(degp_ref):
    deg = degp_ref[pl.ds(0, N)] + degp_ref[pl.ds(NP, N)]
    return lax.rsqrt(deg + 1.0)[:, None]


def _mm_body(obj_ref, rel_ref, wo_ref, wr_ref, degp_ref, y_ref):
    dinv = _dinv(degp_ref)
    y_ref[0] = lax.dot_general(obj_ref[...], wo_ref[...],
                               (((1,), (1,)), ((), ())),
                               preferred_element_type=jnp.float32) * dinv
    y_ref[1] = lax.dot_general(rel_ref[...], wr_ref[...],
                               (((1,), (1,)), ((), ())),
                               preferred_element_type=jnp.float32) * dinv


def _mm(obj_vecs, rel_vecs, W_obj, W_rel, degp):
    return pl.pallas_call(
        _mm_body,
        out_shape=jax.ShapeDtypeStruct((NC, N, D), jnp.float32),
    )(obj_vecs, rel_vecs, W_obj, W_rel, degp)


def _fin_body(acc_ref, y_ref, degp_ref, bo_ref, br_ref, oo_ref, or_ref):
    dinv = _dinv(degp_ref)
    oo_ref[...] = jnp.maximum(
        (acc_ref[0] + y_ref[0]) * dinv + bo_ref[...][None, :], 0.0)
    or_ref[...] = jnp.maximum(
        (acc_ref[1] + y_ref[1]) * dinv + br_ref[...][None, :], 0.0)


def _fin(acc, y, degp, b_obj, b_rel):
    return pl.pallas_call(
        _fin_body,
        out_shape=(jax.ShapeDtypeStruct((N, D), jnp.float32),
                   jax.ShapeDtypeStruct((N, D), jnp.float32)),
    )(acc, y, degp, b_obj, b_rel)


def kernel(obj_vecs, rel_vecs, edge_index, W_obj, b_obj, W_rel, b_rel):
    src = edge_index[:, 0]
    dst = edge_index[:, 1]
    degp = _deg_kernel(dst.reshape(NC, NS, A_CHUNKS, A_B))
    y = _mm(obj_vecs, rel_vecs, W_obj, W_rel, degp)
    pk = jnp.left_shift(dst, 16) | src
    pad = jnp.full((E_PAD - E,), jnp.int32(0xFFFF) << 16, jnp.int32)
    pk_p = jnp.concatenate([pk, pad]).reshape(NS, C_CHUNKS, C_B)
    acc = _scatter_kernel(y, pk_p)
    return _fin(acc, y, degp, b_obj, b_rel)
